# baseline (speedup 1.0000x reference)
import functools

import jax
import jax.numpy as jnp
from jax import lax
from jax.experimental import pallas as pl
from jax.experimental.pallas import tpu as pltpu
from jax.experimental.pallas import tpu_sc as plsc

B, N, P, K = 4, 4096, 1024, 16
IN_CH, OUT_CH, MID, POS_H, HID = 32, 64, 16, 64, 64
NEG = -3.0e38
BIGI = 2 ** 30


def _fps_body(x_ref, nxo_ref, dist_ref):
    liota = jax.lax.broadcasted_iota(jnp.int32, (1, N), 1)
    tiota = (jax.lax.broadcasted_iota(jnp.int32, (8, 128), 0) * 128 +
             jax.lax.broadcasted_iota(jnp.int32, (8, 128), 1))[None]
    dist_ref[...] = jnp.full((B, N), 1e10, jnp.float32)
    x = x_ref[...]
    x0 = x[:, 0]
    x1 = x[:, 1]
    x2 = x[:, 2]

    def step(i, far):
        msk = liota == far
        c0 = jnp.max(jnp.where(msk, x0, NEG), axis=1, keepdims=True)
        c1 = jnp.max(jnp.where(msk, x1, NEG), axis=1, keepdims=True)
        c2 = jnp.max(jnp.where(msk, x2, NEG), axis=1, keepdims=True)
        wmsk = tiota == i
        nxo_ref[:, 0] = jnp.where(wmsk, c0[:, :, None], nxo_ref[:, 0])
        nxo_ref[:, 1] = jnp.where(wmsk, c1[:, :, None], nxo_ref[:, 1])
        nxo_ref[:, 2] = jnp.where(wmsk, c2[:, :, None], nxo_ref[:, 2])
        d = (x0 - c0) ** 2 + (x1 - c1) ** 2 + (x2 - c2) ** 2
        dist = jnp.minimum(dist_ref[...], d)
        dist_ref[...] = dist
        nxt = jnp.argmax(dist, axis=1, keepdims=True).astype(jnp.int32)
        return nxt

    jax.lax.fori_loop(0, P, step, jnp.zeros((B, 1), jnp.int32))


def _fps(xyz):
    nxo = pl.pallas_call(
        _fps_body,
        out_shape=jax.ShapeDtypeStruct((B, 3, 8, 128), jnp.float32),
        scratch_shapes=[pltpu.VMEM((B, N), jnp.float32)],
    )(xyz)
    return nxo.reshape(B, 3, P)


M_ROWS = B * P * K
NW = 32
PER_W = M_ROWS // NW
CH = 128


def _sc_gather48(table, idx):
    mesh = plsc.VectorSubcoreMesh(core_axis_name="c", subcore_axis_name="s")

    @functools.partial(
        pl.kernel,
        out_type=jax.ShapeDtypeStruct((M_ROWS, 48), jnp.float32),
        mesh=mesh,
        scratch_types=[pltpu.VMEM((CH,), jnp.int32),
                       pltpu.VMEM((CH, 48), jnp.float32),
                       pltpu.SemaphoreType.DMA],
        compiler_params=pltpu.CompilerParams(use_tc_tiling_on_sc=False),
    )
    def k(table_hbm, idx_hbm, out_hbm, idx_v, rows_v, sem):
        wid = lax.axis_index("s") * 2 + lax.axis_index("c")
        base = wid * PER_W

        def body(j, c):
            off = base + j * CH
            pltpu.sync_copy(idx_hbm.at[pl.ds(off, CH)], idx_v)
            pltpu.async_copy(table_hbm.at[idx_v], rows_v, sem).wait()
            pltpu.sync_copy(rows_v, out_hbm.at[pl.ds(off, CH)])
            return c

        lax.fori_loop(0, PER_W // CH, body, 0)

    return k(table, idx)


def _knn_body(q_ref, r_ref, o_ref, *, R, QB):
    qx = q_ref[0, :, 0:1]
    qy = q_ref[0, :, 1:2]
    qz = q_ref[0, :, 2:3]
    rx = r_ref[0, 0:1, :]
    ry = r_ref[0, 1:2, :]
    rz = r_ref[0, 2:3, :]
    d = (qx - rx) ** 2 + (qy - ry) ** 2 + (qz - rz) ** 2
    col = jax.lax.broadcasted_iota(jnp.int32, (QB, R), 1)
    cols = []
    for _ in range(K):
        idx = jnp.argmin(d, axis=1, keepdims=True).astype(jnp.int32)
        cols.append(idx)
        d = jnp.where(col == idx, jnp.float32(3.0e38), d)
    o_ref[0] = jnp.concatenate(cols, axis=1) + pl.program_id(0) * R


def _knn(q_rows, refs_cm, R, QB=128):
    nq = q_rows.shape[1]
    grid = (B, nq // QB)
    return pl.pallas_call(
        functools.partial(_knn_body, R=R, QB=QB),
        grid=grid,
        in_specs=[
            pl.BlockSpec((1, QB, 4), lambda b, p: (b, p, 0)),
            pl.BlockSpec((1, 3, R), lambda b, p: (b, 0, 0)),
        ],
        out_specs=pl.BlockSpec((1, QB, K), lambda b, p: (b, p, 0)),
        out_shape=jax.ShapeDtypeStruct((B, nq, K), jnp.int32),
    )(q_rows, refs_cm)


PB_A = 128
RB_A = PB_A * K


def _passa_body(rows_ref, nx_ref, wf_ref, wx_ref, bd_ref, y_ref, st_ref, acc_ref):
    b = pl.program_id(0)
    pb = pl.program_id(1)
    rows = rows_ref[...]
    feat = rows[:, :IN_CH]
    gx = rows[:, IN_CH:IN_CH + 4]
    rel = (gx.reshape(PB_A, K, 4) - nx_ref[0].reshape(PB_A, 1, 4)).reshape(RB_A, 4)
    y = (jnp.dot(feat, wf_ref[...], preferred_element_type=jnp.float32)
         + jnp.dot(rel, wx_ref[...], preferred_element_type=jnp.float32)
         + bd_ref[...])
    y_ref[...] = y

    @pl.when(jnp.logical_and(b == 0, pb == 0))
    def _init():
        acc_ref[...] = jnp.zeros_like(acc_ref)

    acc_ref[0] += jnp.sum(y.reshape(RB_A // 8, 8, OUT_CH), axis=0)
    acc_ref[1] += jnp.sum((y * y).reshape(RB_A // 8, 8, OUT_CH), axis=0)

    @pl.when(jnp.logical_and(b == B - 1, pb == pl.num_programs(1) - 1))
    def _fin():
        st_ref[...] = acc_ref[...]


def _passa(g1rows, nxyz_rows, wf_t, wx_t, b_down):
    grid = (B, P // PB_A)
    nb = P // PB_A
    return pl.pallas_call(
        _passa_body,
        grid=grid,
        in_specs=[
            pl.BlockSpec((RB_A, 48), lambda b, p: (b * nb + p, 0)),
            pl.BlockSpec((1, PB_A, 4), lambda b, p: (b, p, 0)),
            pl.BlockSpec((IN_CH, OUT_CH), lambda b, p: (0, 0)),
            pl.BlockSpec((4, OUT_CH), lambda b, p: (0, 0)),
            pl.BlockSpec((1, OUT_CH), lambda b, p: (0, 0)),
        ],
        out_specs=[
            pl.BlockSpec((RB_A, OUT_CH), lambda b, p: (b * nb + p, 0)),
            pl.BlockSpec((2, 8, OUT_CH), lambda b, p: (0, 0, 0)),
        ],
        out_shape=[
            jax.ShapeDtypeStruct((B * P * K, OUT_CH), jnp.float32),
            jax.ShapeDtypeStruct((2, 8, OUT_CH), jnp.float32),
        ],
        scratch_shapes=[pltpu.VMEM((2, 8, OUT_CH), jnp.float32)],
    )(g1rows, nxyz_rows, wf_t, wx_t, b_down)


PB_B = 128
RB_B = PB_B * K
MTOT = float(B * P * K)


def _passb_body(y_ref, st_ref, g_ref, be_ref, wb_ref, bb_ref,
                wq_ref, wk_ref, wv_ref, np_ref, q_ref, tk_ref, tv_ref):
    mean = jnp.sum(st_ref[0], axis=0, keepdims=True) * (1.0 / MTOT)
    var = jnp.sum(st_ref[1], axis=0, keepdims=True) * (1.0 / MTOT) - mean * mean
    y = y_ref[...]
    xh = (y - mean) / jnp.sqrt(var + 1e-5)
    h = jnp.maximum(xh * g_ref[...] + be_ref[...], 0.0)
    npts = jnp.max(h.reshape(PB_B, K, OUT_CH), axis=1)
    np_ref[0] = npts
    pts = jnp.dot(npts, wb_ref[...], preferred_element_type=jnp.float32) + bb_ref[...]
    q_ref[0] = jnp.dot(pts, wq_ref[...], preferred_element_type=jnp.float32)
    tk_ref[0] = jnp.dot(pts, wk_ref[...], preferred_element_type=jnp.float32)
    tv_ref[0] = jnp.dot(pts, wv_ref[...], preferred_element_type=jnp.float32)


def _passb(y, stats, gamma, beta, wb_t, bb, wq_t, wk_t, wv_t):
    grid = (B, P // PB_B)
    o16 = lambda: pl.BlockSpec((1, PB_B, MID), lambda b, p: (b, p, 0))
    s16 = lambda: jax.ShapeDtypeStruct((B, P, MID), jnp.float32)
    return pl.pallas_call(
        _passb_body,
        grid=grid,
        in_specs=[
            pl.BlockSpec((RB_B, OUT_CH), lambda b, p: (b * (P // PB_B) + p, 0)),
            pl.BlockSpec((2, 8, OUT_CH), lambda b, p: (0, 0, 0)),
            pl.BlockSpec((1, OUT_CH), lambda b, p: (0, 0)),
            pl.BlockSpec((1, OUT_CH), lambda b, p: (0, 0)),
            pl.BlockSpec((OUT_CH, MID), lambda b, p: (0, 0)),
            pl.BlockSpec((1, MID), lambda b, p: (0, 0)),
            pl.BlockSpec((MID, MID), lambda b, p: (0, 0)),
            pl.BlockSpec((MID, MID), lambda b, p: (0, 0)),
            pl.BlockSpec((MID, MID), lambda b, p: (0, 0)),
        ],
        out_specs=[
            pl.BlockSpec((1, PB_B, OUT_CH), lambda b, p: (b, p, 0)),
            o16(), o16(), o16(),
        ],
        out_shape=[
            jax.ShapeDtypeStruct((B, P, OUT_CH), jnp.float32),
            s16(), s16(), s16(),
        ],
    )(y, stats, gamma, beta, wb_t, bb, wq_t, wk_t, wv_t)


PB_T = 128
RB_T = PB_T * K


def _attn_body(rows_ref, q_ref, cx_ref, np_ref,
               wp1_ref, bp1_ref, wp2_ref, bp2_ref,
               wa1_ref, ba1_ref, wa2_ref, ba2_ref, wa_ref, ba_ref, o_ref):
    rows = rows_ref[...]
    k_n = rows[:, :MID]
    v_n = rows[:, MID:2 * MID]
    x_n = rows[:, 2 * MID:2 * MID + 4]
    rel = (cx_ref[0].reshape(PB_T, 1, 4) - x_n.reshape(PB_T, K, 4)).reshape(RB_T, 4)
    rp = jnp.maximum(
        jnp.dot(rel, wp1_ref[...], preferred_element_type=jnp.float32) + bp1_ref[...], 0.0)
    rp = jnp.dot(rp, wp2_ref[...], preferred_element_type=jnp.float32) + bp2_ref[...]
    qk = (q_ref[0].reshape(PB_T, 1, MID) - k_n.reshape(PB_T, K, MID)).reshape(RB_T, MID)
    u = qk + rp
    s = jnp.maximum(
        jnp.dot(u, wa1_ref[...], preferred_element_type=jnp.float32) + ba1_ref[...], 0.0)
    s = jnp.dot(s, wa2_ref[...], preferred_element_type=jnp.float32) + ba2_ref[...]
    s3 = s.reshape(PB_T, K, MID)
    m = jnp.max(s3, axis=1, keepdims=True)
    e = jnp.exp(s3 - m)
    attn = e / jnp.sum(e, axis=1, keepdims=True)
    v2 = (v_n + rp).reshape(PB_T, K, MID)
    agg = jnp.sum(attn * v2, axis=1)
    o_ref[0] = np_ref[0] + jnp.dot(agg, wa_ref[...],
                                   preferred_element_type=jnp.float32) + ba_ref[...]


def _attn(g2rows, nxyz_rows, q, np_rows,
          wp1_t, bp1, wp2_t, bp2, wa1_t, ba1, wa2_t, ba2, wa_t, ba):
    grid = (B, P // PB_T)
    nb = P // PB_T
    w = lambda r, c: pl.BlockSpec((r, c), lambda b, p: (0, 0))
    return pl.pallas_call(
        _attn_body,
        grid=grid,
        in_specs=[
            pl.BlockSpec((RB_T, 48), lambda b, p: (b * nb + p, 0)),
            pl.BlockSpec((1, PB_T, MID), lambda b, p: (b, p, 0)),
            pl.BlockSpec((1, PB_T, 4), lambda b, p: (b, p, 0)),
            pl.BlockSpec((1, PB_T, OUT_CH), lambda b, p: (b, p, 0)),
            w(4, POS_H), w(1, POS_H), w(POS_H, MID), w(1, MID),
            w(MID, HID), w(1, HID), w(HID, MID), w(1, MID),
            w(MID, OUT_CH), w(1, OUT_CH),
        ],
        out_specs=pl.BlockSpec((1, PB_T, OUT_CH), lambda b, p: (b, p, 0)),
        out_shape=jax.ShapeDtypeStruct((B, P, OUT_CH), jnp.float32),
    )(g2rows, q, nxyz_rows, np_rows,
      wp1_t, bp1, wp2_t, bp2, wa1_t, ba1, wa2_t, ba2, wa_t, ba)


def kernel(xyz, points, W_down, b_down, bn_gamma, bn_beta, Wb, bb, Wqkv,
           Wp1, bp1, Wp2, bp2, Wa1, ba1, Wa2, ba2, Wa, ba):
    wf_t = jnp.transpose(W_down[:, :IN_CH])
    wx_t = jnp.concatenate(
        [jnp.transpose(W_down[:, IN_CH:]), jnp.zeros((1, OUT_CH))], axis=0)
    bd = b_down.reshape(1, OUT_CH)
    wb_t = jnp.transpose(Wb)
    wq_t = jnp.transpose(Wqkv[:MID])
    wk_t = jnp.transpose(Wqkv[MID:2 * MID])
    wv_t = jnp.transpose(Wqkv[2 * MID:])
    wp1_t = jnp.concatenate([jnp.transpose(Wp1), jnp.zeros((1, POS_H))], axis=0)
    wp2_t = jnp.transpose(Wp2)
    wa1_t = jnp.transpose(Wa1)
    wa2_t = jnp.transpose(Wa2)
    wa_t = jnp.transpose(Wa)

    new_xyz = _fps(xyz)
    nxyz_rows = jnp.concatenate(
        [jnp.transpose(new_xyz, (0, 2, 1)), jnp.zeros((B, P, 1))], axis=2)
    gidx1 = _knn(nxyz_rows, xyz, R=N)
    t1 = jnp.concatenate(
        [jnp.transpose(points, (0, 2, 1)),
         jnp.transpose(xyz, (0, 2, 1)),
         jnp.zeros((B, N, 13))], axis=2).reshape(B * N, 48)
    g1rows = _sc_gather48(t1, gidx1.reshape(-1))
    y, stats = _passa(g1rows, nxyz_rows, wf_t, wx_t, bd)
    np_rows, q, t2k, t2v = _passb(
        y, stats, bn_gamma.reshape(1, -1), bn_beta.reshape(1, -1),
        wb_t, bb.reshape(1, -1), wq_t, wk_t, wv_t)
    gidx2 = _knn(nxyz_rows, new_xyz, R=P)
    t2 = jnp.concatenate(
        [t2k, t2v, nxyz_rows, jnp.zeros((B, P, 12))], axis=2).reshape(B * P, 48)
    g2rows = _sc_gather48(t2, gidx2.reshape(-1))
    out_rows = _attn(g2rows, nxyz_rows, q, np_rows,
                     wp1_t, bp1.reshape(1, -1), wp2_t, bp2.reshape(1, -1),
                     wa1_t, ba1.reshape(1, -1), wa2_t, ba2.reshape(1, -1),
                     wa_t, ba.reshape(1, -1))
    return (new_xyz, jnp.transpose(out_rows, (0, 2, 1)))


# speedup vs baseline: 1.1229x; 1.1229x over previous
import functools

import jax
import jax.numpy as jnp
from jax import lax
from jax.experimental import pallas as pl
from jax.experimental.pallas import tpu as pltpu
from jax.experimental.pallas import tpu_sc as plsc

B, N, P, K = 4, 4096, 1024, 16
IN_CH, OUT_CH, MID, POS_H, HID = 32, 64, 16, 64, 64
NEG = -3.0e38
BIGI = 2 ** 30


def _fps_body(x_ref, nxo_ref):
    fiota = (jax.lax.broadcasted_iota(jnp.int32, (8, 512), 0) * 512 +
             jax.lax.broadcasted_iota(jnp.int32, (8, 512), 1))[None]
    tiota = (jax.lax.broadcasted_iota(jnp.int32, (8, 128), 0) * 128 +
             jax.lax.broadcasted_iota(jnp.int32, (8, 128), 1))[None]
    x = x_ref[...]
    x0 = x[:, 0]
    x1 = x[:, 1]
    x2 = x[:, 2]

    def step(i, carry):
        far, dist0 = carry
        msk = fiota == far
        c0 = jnp.max(jnp.where(msk, x0, NEG), axis=(1, 2), keepdims=True)
        c1 = jnp.max(jnp.where(msk, x1, NEG), axis=(1, 2), keepdims=True)
        c2 = jnp.max(jnp.where(msk, x2, NEG), axis=(1, 2), keepdims=True)
        wmsk = tiota == i
        nxo_ref[:, 0] = jnp.where(wmsk, c0, nxo_ref[:, 0])
        nxo_ref[:, 1] = jnp.where(wmsk, c1, nxo_ref[:, 1])
        nxo_ref[:, 2] = jnp.where(wmsk, c2, nxo_ref[:, 2])
        d = (x0 - c0) ** 2 + (x1 - c1) ** 2 + (x2 - c2) ** 2
        dist = jnp.minimum(dist0, d)
        m = jnp.max(dist, axis=(1, 2), keepdims=True)
        nxt = jnp.min(jnp.where(dist == m, fiota, BIGI), axis=(1, 2), keepdims=True)
        return (nxt, dist)

    jax.lax.fori_loop(
        0, P, step,
        (jnp.zeros((B, 1, 1), jnp.int32), jnp.full((B, 8, 512), 1e10, jnp.float32)))


def _fps(xyz):
    x_r = xyz.reshape(B, 3, 8, 512)
    nxo = pl.pallas_call(
        _fps_body,
        out_shape=jax.ShapeDtypeStruct((B, 3, 8, 128), jnp.float32),
    )(x_r)
    return nxo.reshape(B, 3, P)


M_ROWS = B * P * K
NW = 32
PER_W = M_ROWS // NW
CH = 128


def _sc_gather48(table, idx):
    mesh = plsc.VectorSubcoreMesh(core_axis_name="c", subcore_axis_name="s")

    @functools.partial(
        pl.kernel,
        out_type=jax.ShapeDtypeStruct((M_ROWS, 48), jnp.float32),
        mesh=mesh,
        scratch_types=[pltpu.VMEM((CH,), jnp.int32),
                       pltpu.VMEM((CH, 48), jnp.float32),
                       pltpu.SemaphoreType.DMA],
        compiler_params=pltpu.CompilerParams(use_tc_tiling_on_sc=False),
    )
    def k(table_hbm, idx_hbm, out_hbm, idx_v, rows_v, sem):
        wid = lax.axis_index("s") * 2 + lax.axis_index("c")
        base = wid * PER_W

        def body(j, c):
            off = base + j * CH
            pltpu.sync_copy(idx_hbm.at[pl.ds(off, CH)], idx_v)
            pltpu.async_copy(table_hbm.at[idx_v], rows_v, sem).wait()
            pltpu.sync_copy(rows_v, out_hbm.at[pl.ds(off, CH)])
            return c

        lax.fori_loop(0, PER_W // CH, body, 0)

    return k(table, idx)


def _knn_body(q_ref, r_ref, o_ref, *, R, QB):
    qx = q_ref[0, :, 0:1]
    qy = q_ref[0, :, 1:2]
    qz = q_ref[0, :, 2:3]
    rx = r_ref[0, 0:1, :]
    ry = r_ref[0, 1:2, :]
    rz = r_ref[0, 2:3, :]
    d = (qx - rx) ** 2 + (qy - ry) ** 2 + (qz - rz) ** 2
    col = jax.lax.broadcasted_iota(jnp.int32, (QB, R), 1)
    cols = []
    for _ in range(K):
        idx = jnp.argmin(d, axis=1, keepdims=True).astype(jnp.int32)
        cols.append(idx)
        d = jnp.where(col == idx, jnp.float32(3.0e38), d)
    o_ref[0] = jnp.concatenate(cols, axis=1) + pl.program_id(0) * R


def _knn(q_rows, refs_cm, R, QB=128):
    nq = q_rows.shape[1]
    grid = (B, nq // QB)
    return pl.pallas_call(
        functools.partial(_knn_body, R=R, QB=QB),
        grid=grid,
        in_specs=[
            pl.BlockSpec((1, QB, 4), lambda b, p: (b, p, 0)),
            pl.BlockSpec((1, 3, R), lambda b, p: (b, 0, 0)),
        ],
        out_specs=pl.BlockSpec((1, QB, K), lambda b, p: (b, p, 0)),
        out_shape=jax.ShapeDtypeStruct((B, nq, K), jnp.int32),
    )(q_rows, refs_cm)


PB_A = 128
RB_A = PB_A * K


def _passa_body(rows_ref, nx_ref, wf_ref, wx_ref, bd_ref, y_ref, st_ref, acc_ref):
    b = pl.program_id(0)
    pb = pl.program_id(1)
    rows = rows_ref[...]
    feat = rows[:, :IN_CH]
    gx = rows[:, IN_CH:IN_CH + 4]
    rel = (gx.reshape(PB_A, K, 4) - nx_ref[0].reshape(PB_A, 1, 4)).reshape(RB_A, 4)
    y = (jnp.dot(feat, wf_ref[...], preferred_element_type=jnp.float32)
         + jnp.dot(rel, wx_ref[...], preferred_element_type=jnp.float32)
         + bd_ref[...])
    y_ref[...] = y

    @pl.when(jnp.logical_and(b == 0, pb == 0))
    def _init():
        acc_ref[...] = jnp.zeros_like(acc_ref)

    acc_ref[0] += jnp.sum(y.reshape(RB_A // 8, 8, OUT_CH), axis=0)
    acc_ref[1] += jnp.sum((y * y).reshape(RB_A // 8, 8, OUT_CH), axis=0)

    @pl.when(jnp.logical_and(b == B - 1, pb == pl.num_programs(1) - 1))
    def _fin():
        st_ref[...] = acc_ref[...]


def _passa(g1rows, nxyz_rows, wf_t, wx_t, b_down):
    grid = (B, P // PB_A)
    nb = P // PB_A
    return pl.pallas_call(
        _passa_body,
        grid=grid,
        in_specs=[
            pl.BlockSpec((RB_A, 48), lambda b, p: (b * nb + p, 0)),
            pl.BlockSpec((1, PB_A, 4), lambda b, p: (b, p, 0)),
            pl.BlockSpec((IN_CH, OUT_CH), lambda b, p: (0, 0)),
            pl.BlockSpec((4, OUT_CH), lambda b, p: (0, 0)),
            pl.BlockSpec((1, OUT_CH), lambda b, p: (0, 0)),
        ],
        out_specs=[
            pl.BlockSpec((RB_A, OUT_CH), lambda b, p: (b * nb + p, 0)),
            pl.BlockSpec((2, 8, OUT_CH), lambda b, p: (0, 0, 0)),
        ],
        out_shape=[
            jax.ShapeDtypeStruct((B * P * K, OUT_CH), jnp.float32),
            jax.ShapeDtypeStruct((2, 8, OUT_CH), jnp.float32),
        ],
        scratch_shapes=[pltpu.VMEM((2, 8, OUT_CH), jnp.float32)],
    )(g1rows, nxyz_rows, wf_t, wx_t, b_down)


PB_B = 128
RB_B = PB_B * K
MTOT = float(B * P * K)


def _passb_body(y_ref, st_ref, g_ref, be_ref, wb_ref, bb_ref,
                wq_ref, wk_ref, wv_ref, np_ref, q_ref, tk_ref, tv_ref):
    mean = jnp.sum(st_ref[0], axis=0, keepdims=True) * (1.0 / MTOT)
    var = jnp.sum(st_ref[1], axis=0, keepdims=True) * (1.0 / MTOT) - mean * mean
    y = y_ref[...]
    xh = (y - mean) / jnp.sqrt(var + 1e-5)
    h = jnp.maximum(xh * g_ref[...] + be_ref[...], 0.0)
    npts = jnp.max(h.reshape(PB_B, K, OUT_CH), axis=1)
    np_ref[0] = npts
    pts = jnp.dot(npts, wb_ref[...], preferred_element_type=jnp.float32) + bb_ref[...]
    q_ref[0] = jnp.dot(pts, wq_ref[...], preferred_element_type=jnp.float32)
    tk_ref[0] = jnp.dot(pts, wk_ref[...], preferred_element_type=jnp.float32)
    tv_ref[0] = jnp.dot(pts, wv_ref[...], preferred_element_type=jnp.float32)


def _passb(y, stats, gamma, beta, wb_t, bb, wq_t, wk_t, wv_t):
    grid = (B, P // PB_B)
    o16 = lambda: pl.BlockSpec((1, PB_B, MID), lambda b, p: (b, p, 0))
    s16 = lambda: jax.ShapeDtypeStruct((B, P, MID), jnp.float32)
    return pl.pallas_call(
        _passb_body,
        grid=grid,
        in_specs=[
            pl.BlockSpec((RB_B, OUT_CH), lambda b, p: (b * (P // PB_B) + p, 0)),
            pl.BlockSpec((2, 8, OUT_CH), lambda b, p: (0, 0, 0)),
            pl.BlockSpec((1, OUT_CH), lambda b, p: (0, 0)),
            pl.BlockSpec((1, OUT_CH), lambda b, p: (0, 0)),
            pl.BlockSpec((OUT_CH, MID), lambda b, p: (0, 0)),
            pl.BlockSpec((1, MID), lambda b, p: (0, 0)),
            pl.BlockSpec((MID, MID), lambda b, p: (0, 0)),
            pl.BlockSpec((MID, MID), lambda b, p: (0, 0)),
            pl.BlockSpec((MID, MID), lambda b, p: (0, 0)),
        ],
        out_specs=[
            pl.BlockSpec((1, PB_B, OUT_CH), lambda b, p: (b, p, 0)),
            o16(), o16(), o16(),
        ],
        out_shape=[
            jax.ShapeDtypeStruct((B, P, OUT_CH), jnp.float32),
            s16(), s16(), s16(),
        ],
    )(y, stats, gamma, beta, wb_t, bb, wq_t, wk_t, wv_t)


PB_T = 128
RB_T = PB_T * K


def _attn_body(rows_ref, q_ref, cx_ref, np_ref,
               wp1_ref, bp1_ref, wp2_ref, bp2_ref,
               wa1_ref, ba1_ref, wa2_ref, ba2_ref, wa_ref, ba_ref, o_ref):
    rows = rows_ref[...]
    k_n = rows[:, :MID]
    v_n = rows[:, MID:2 * MID]
    x_n = rows[:, 2 * MID:2 * MID + 4]
    rel = (cx_ref[0].reshape(PB_T, 1, 4) - x_n.reshape(PB_T, K, 4)).reshape(RB_T, 4)
    rp = jnp.maximum(
        jnp.dot(rel, wp1_ref[...], preferred_element_type=jnp.float32) + bp1_ref[...], 0.0)
    rp = jnp.dot(rp, wp2_ref[...], preferred_element_type=jnp.float32) + bp2_ref[...]
    qk = (q_ref[0].reshape(PB_T, 1, MID) - k_n.reshape(PB_T, K, MID)).reshape(RB_T, MID)
    u = qk + rp
    s = jnp.maximum(
        jnp.dot(u, wa1_ref[...], preferred_element_type=jnp.float32) + ba1_ref[...], 0.0)
    s = jnp.dot(s, wa2_ref[...], preferred_element_type=jnp.float32) + ba2_ref[...]
    s3 = s.reshape(PB_T, K, MID)
    m = jnp.max(s3, axis=1, keepdims=True)
    e = jnp.exp(s3 - m)
    attn = e / jnp.sum(e, axis=1, keepdims=True)
    v2 = (v_n + rp).reshape(PB_T, K, MID)
    agg = jnp.sum(attn * v2, axis=1)
    o_ref[0] = np_ref[0] + jnp.dot(agg, wa_ref[...],
                                   preferred_element_type=jnp.float32) + ba_ref[...]


def _attn(g2rows, nxyz_rows, q, np_rows,
          wp1_t, bp1, wp2_t, bp2, wa1_t, ba1, wa2_t, ba2, wa_t, ba):
    grid = (B, P // PB_T)
    nb = P // PB_T
    w = lambda r, c: pl.BlockSpec((r, c), lambda b, p: (0, 0))
    return pl.pallas_call(
        _attn_body,
        grid=grid,
        in_specs=[
            pl.BlockSpec((RB_T, 48), lambda b, p: (b * nb + p, 0)),
            pl.BlockSpec((1, PB_T, MID), lambda b, p: (b, p, 0)),
            pl.BlockSpec((1, PB_T, 4), lambda b, p: (b, p, 0)),
            pl.BlockSpec((1, PB_T, OUT_CH), lambda b, p: (b, p, 0)),
            w(4, POS_H), w(1, POS_H), w(POS_H, MID), w(1, MID),
            w(MID, HID), w(1, HID), w(HID, MID), w(1, MID),
            w(MID, OUT_CH), w(1, OUT_CH),
        ],
        out_specs=pl.BlockSpec((1, PB_T, OUT_CH), lambda b, p: (b, p, 0)),
        out_shape=jax.ShapeDtypeStruct((B, P, OUT_CH), jnp.float32),
    )(g2rows, q, nxyz_rows, np_rows,
      wp1_t, bp1, wp2_t, bp2, wa1_t, ba1, wa2_t, ba2, wa_t, ba)


def kernel(xyz, points, W_down, b_down, bn_gamma, bn_beta, Wb, bb, Wqkv,
           Wp1, bp1, Wp2, bp2, Wa1, ba1, Wa2, ba2, Wa, ba):
    wf_t = jnp.transpose(W_down[:, :IN_CH])
    wx_t = jnp.concatenate(
        [jnp.transpose(W_down[:, IN_CH:]), jnp.zeros((1, OUT_CH))], axis=0)
    bd = b_down.reshape(1, OUT_CH)
    wb_t = jnp.transpose(Wb)
    wq_t = jnp.transpose(Wqkv[:MID])
    wk_t = jnp.transpose(Wqkv[MID:2 * MID])
    wv_t = jnp.transpose(Wqkv[2 * MID:])
    wp1_t = jnp.concatenate([jnp.transpose(Wp1), jnp.zeros((1, POS_H))], axis=0)
    wp2_t = jnp.transpose(Wp2)
    wa1_t = jnp.transpose(Wa1)
    wa2_t = jnp.transpose(Wa2)
    wa_t = jnp.transpose(Wa)

    new_xyz = _fps(xyz)
    nxyz_rows = jnp.concatenate(
        [jnp.transpose(new_xyz, (0, 2, 1)), jnp.zeros((B, P, 1))], axis=2)
    gidx1 = _knn(nxyz_rows, xyz, R=N)
    t1 = jnp.concatenate(
        [jnp.transpose(points, (0, 2, 1)),
         jnp.transpose(xyz, (0, 2, 1)),
         jnp.zeros((B, N, 13))], axis=2).reshape(B * N, 48)
    g1rows = _sc_gather48(t1, gidx1.reshape(-1))
    y, stats = _passa(g1rows, nxyz_rows, wf_t, wx_t, bd)
    np_rows, q, t2k, t2v = _passb(
        y, stats, bn_gamma.reshape(1, -1), bn_beta.reshape(1, -1),
        wb_t, bb.reshape(1, -1), wq_t, wk_t, wv_t)
    gidx2 = _knn(nxyz_rows, new_xyz, R=P)
    t2 = jnp.concatenate(
        [t2k, t2v, nxyz_rows, jnp.zeros((B, P, 12))], axis=2).reshape(B * P, 48)
    g2rows = _sc_gather48(t2, gidx2.reshape(-1))
    out_rows = _attn(g2rows, nxyz_rows, q, np_rows,
                     wp1_t, bp1.reshape(1, -1), wp2_t, bp2.reshape(1, -1),
                     wa1_t, ba1.reshape(1, -1), wa2_t, ba2.reshape(1, -1),
                     wa_t, ba.reshape(1, -1))
    return (new_xyz, jnp.transpose(out_rows, (0, 2, 1)))


# speedup vs baseline: 1.2393x; 1.1037x over previous
import functools

import jax
import jax.numpy as jnp
from jax import lax
from jax.experimental import pallas as pl
from jax.experimental.pallas import tpu as pltpu
from jax.experimental.pallas import tpu_sc as plsc

B, N, P, K = 4, 4096, 1024, 16
IN_CH, OUT_CH, MID, POS_H, HID = 32, 64, 16, 64, 64
NEG = -3.0e38
BIGI = 2 ** 30


def _fps_body(x_ref, nxo_ref):
    fiota = (jax.lax.broadcasted_iota(jnp.int32, (8, 512), 0) * 512 +
             jax.lax.broadcasted_iota(jnp.int32, (8, 512), 1))[None]
    tiota = (jax.lax.broadcasted_iota(jnp.int32, (8, 128), 0) * 128 +
             jax.lax.broadcasted_iota(jnp.int32, (8, 128), 1))[None]
    x = x_ref[...]
    x0 = x[:, 0]
    x1 = x[:, 1]
    x2 = x[:, 2]

    siota = jax.lax.broadcasted_iota(jnp.int32, (1, 8, 1), 1)

    def step(i, carry):
        far, dist0 = carry
        msk = fiota == far
        c0 = jnp.max(jnp.max(jnp.where(msk, x0, NEG), axis=2, keepdims=True),
                     axis=1, keepdims=True)
        c1 = jnp.max(jnp.max(jnp.where(msk, x1, NEG), axis=2, keepdims=True),
                     axis=1, keepdims=True)
        c2 = jnp.max(jnp.max(jnp.where(msk, x2, NEG), axis=2, keepdims=True),
                     axis=1, keepdims=True)
        wmsk = tiota == i
        nxo_ref[:, 0] = jnp.where(wmsk, c0, nxo_ref[:, 0])
        nxo_ref[:, 1] = jnp.where(wmsk, c1, nxo_ref[:, 1])
        nxo_ref[:, 2] = jnp.where(wmsk, c2, nxo_ref[:, 2])
        d = (x0 - c0) ** 2 + (x1 - c1) ** 2 + (x2 - c2) ** 2
        dist = jnp.minimum(dist0, d)
        m1 = jnp.max(dist, axis=2, keepdims=True)
        i1 = jnp.argmax(dist, axis=2, keepdims=True).astype(jnp.int32)
        m = jnp.max(m1, axis=1, keepdims=True)
        nxt_s = jnp.min(jnp.where(m1 == m, siota, BIGI), axis=1, keepdims=True)
        nxt_c = jnp.max(jnp.where(siota == nxt_s, i1, 0), axis=1, keepdims=True)
        nxt = nxt_s * 512 + nxt_c
        return (nxt, dist)

    jax.lax.fori_loop(
        0, P, step,
        (jnp.zeros((B, 1, 1), jnp.int32), jnp.full((B, 8, 512), 1e10, jnp.float32)))


def _fps(xyz):
    x_r = xyz.reshape(B, 3, 8, 512)
    nxo = pl.pallas_call(
        _fps_body,
        out_shape=jax.ShapeDtypeStruct((B, 3, 8, 128), jnp.float32),
    )(x_r)
    return nxo.reshape(B, 3, P)


M_ROWS = B * P * K
NW = 32
PER_W = M_ROWS // NW
CH = 128


def _sc_gather48(table, idx):
    mesh = plsc.VectorSubcoreMesh(core_axis_name="c", subcore_axis_name="s")

    @functools.partial(
        pl.kernel,
        out_type=jax.ShapeDtypeStruct((M_ROWS, 48), jnp.float32),
        mesh=mesh,
        scratch_types=[pltpu.VMEM((CH,), jnp.int32),
                       pltpu.VMEM((CH, 48), jnp.float32),
                       pltpu.SemaphoreType.DMA],
        compiler_params=pltpu.CompilerParams(use_tc_tiling_on_sc=False),
    )
    def k(table_hbm, idx_hbm, out_hbm, idx_v, rows_v, sem):
        wid = lax.axis_index("s") * 2 + lax.axis_index("c")
        base = wid * PER_W

        def body(j, c):
            off = base + j * CH
            pltpu.sync_copy(idx_hbm.at[pl.ds(off, CH)], idx_v)
            pltpu.async_copy(table_hbm.at[idx_v], rows_v, sem).wait()
            pltpu.sync_copy(rows_v, out_hbm.at[pl.ds(off, CH)])
            return c

        lax.fori_loop(0, PER_W // CH, body, 0)

    return k(table, idx)


def _knn_body(q_ref, r_ref, o_ref, *, R, QB):
    qx = q_ref[0, :, 0:1]
    qy = q_ref[0, :, 1:2]
    qz = q_ref[0, :, 2:3]
    rx = r_ref[0, 0:1, :]
    ry = r_ref[0, 1:2, :]
    rz = r_ref[0, 2:3, :]
    d = (qx - rx) ** 2 + (qy - ry) ** 2 + (qz - rz) ** 2
    col = jax.lax.broadcasted_iota(jnp.int32, (QB, R), 1)
    cols = []
    for _ in range(K):
        idx = jnp.argmin(d, axis=1, keepdims=True).astype(jnp.int32)
        cols.append(idx)
        d = jnp.where(col == idx, jnp.float32(3.0e38), d)
    o_ref[0] = jnp.concatenate(cols, axis=1) + pl.program_id(0) * R


def _knn(q_rows, refs_cm, R, QB=128):
    nq = q_rows.shape[1]
    grid = (B, nq // QB)
    return pl.pallas_call(
        functools.partial(_knn_body, R=R, QB=QB),
        grid=grid,
        in_specs=[
            pl.BlockSpec((1, QB, 4), lambda b, p: (b, p, 0)),
            pl.BlockSpec((1, 3, R), lambda b, p: (b, 0, 0)),
        ],
        out_specs=pl.BlockSpec((1, QB, K), lambda b, p: (b, p, 0)),
        out_shape=jax.ShapeDtypeStruct((B, nq, K), jnp.int32),
    )(q_rows, refs_cm)


PB_A = 128
RB_A = PB_A * K


def _passa_body(rows_ref, nx_ref, wf_ref, wx_ref, bd_ref, y_ref, st_ref, acc_ref):
    b = pl.program_id(0)
    pb = pl.program_id(1)
    rows = rows_ref[...]
    feat = rows[:, :IN_CH]
    gx = rows[:, IN_CH:IN_CH + 4]
    rel = (gx.reshape(PB_A, K, 4) - nx_ref[0].reshape(PB_A, 1, 4)).reshape(RB_A, 4)
    y = (jnp.dot(feat, wf_ref[...], preferred_element_type=jnp.float32)
         + jnp.dot(rel, wx_ref[...], preferred_element_type=jnp.float32)
         + bd_ref[...])
    y_ref[...] = y

    @pl.when(jnp.logical_and(b == 0, pb == 0))
    def _init():
        acc_ref[...] = jnp.zeros_like(acc_ref)

    acc_ref[0] += jnp.sum(y.reshape(RB_A // 8, 8, OUT_CH), axis=0)
    acc_ref[1] += jnp.sum((y * y).reshape(RB_A // 8, 8, OUT_CH), axis=0)

    @pl.when(jnp.logical_and(b == B - 1, pb == pl.num_programs(1) - 1))
    def _fin():
        st_ref[...] = acc_ref[...]


def _passa(g1rows, nxyz_rows, wf_t, wx_t, b_down):
    grid = (B, P // PB_A)
    nb = P // PB_A
    return pl.pallas_call(
        _passa_body,
        grid=grid,
        in_specs=[
            pl.BlockSpec((RB_A, 48), lambda b, p: (b * nb + p, 0)),
            pl.BlockSpec((1, PB_A, 4), lambda b, p: (b, p, 0)),
            pl.BlockSpec((IN_CH, OUT_CH), lambda b, p: (0, 0)),
            pl.BlockSpec((4, OUT_CH), lambda b, p: (0, 0)),
            pl.BlockSpec((1, OUT_CH), lambda b, p: (0, 0)),
        ],
        out_specs=[
            pl.BlockSpec((RB_A, OUT_CH), lambda b, p: (b * nb + p, 0)),
            pl.BlockSpec((2, 8, OUT_CH), lambda b, p: (0, 0, 0)),
        ],
        out_shape=[
            jax.ShapeDtypeStruct((B * P * K, OUT_CH), jnp.float32),
            jax.ShapeDtypeStruct((2, 8, OUT_CH), jnp.float32),
        ],
        scratch_shapes=[pltpu.VMEM((2, 8, OUT_CH), jnp.float32)],
    )(g1rows, nxyz_rows, wf_t, wx_t, b_down)


PB_B = 128
RB_B = PB_B * K
MTOT = float(B * P * K)


def _passb_body(y_ref, st_ref, g_ref, be_ref, wb_ref, bb_ref,
                wq_ref, wk_ref, wv_ref, np_ref, q_ref, tk_ref, tv_ref):
    mean = jnp.sum(st_ref[0], axis=0, keepdims=True) * (1.0 / MTOT)
    var = jnp.sum(st_ref[1], axis=0, keepdims=True) * (1.0 / MTOT) - mean * mean
    y = y_ref[...]
    xh = (y - mean) / jnp.sqrt(var + 1e-5)
    h = jnp.maximum(xh * g_ref[...] + be_ref[...], 0.0)
    npts = jnp.max(h.reshape(PB_B, K, OUT_CH), axis=1)
    np_ref[0] = npts
    pts = jnp.dot(npts, wb_ref[...], preferred_element_type=jnp.float32) + bb_ref[...]
    q_ref[0] = jnp.dot(pts, wq_ref[...], preferred_element_type=jnp.float32)
    tk_ref[0] = jnp.dot(pts, wk_ref[...], preferred_element_type=jnp.float32)
    tv_ref[0] = jnp.dot(pts, wv_ref[...], preferred_element_type=jnp.float32)


def _passb(y, stats, gamma, beta, wb_t, bb, wq_t, wk_t, wv_t):
    grid = (B, P // PB_B)
    o16 = lambda: pl.BlockSpec((1, PB_B, MID), lambda b, p: (b, p, 0))
    s16 = lambda: jax.ShapeDtypeStruct((B, P, MID), jnp.float32)
    return pl.pallas_call(
        _passb_body,
        grid=grid,
        in_specs=[
            pl.BlockSpec((RB_B, OUT_CH), lambda b, p: (b * (P // PB_B) + p, 0)),
            pl.BlockSpec((2, 8, OUT_CH), lambda b, p: (0, 0, 0)),
            pl.BlockSpec((1, OUT_CH), lambda b, p: (0, 0)),
            pl.BlockSpec((1, OUT_CH), lambda b, p: (0, 0)),
            pl.BlockSpec((OUT_CH, MID), lambda b, p: (0, 0)),
            pl.BlockSpec((1, MID), lambda b, p: (0, 0)),
            pl.BlockSpec((MID, MID), lambda b, p: (0, 0)),
            pl.BlockSpec((MID, MID), lambda b, p: (0, 0)),
            pl.BlockSpec((MID, MID), lambda b, p: (0, 0)),
        ],
        out_specs=[
            pl.BlockSpec((1, PB_B, OUT_CH), lambda b, p: (b, p, 0)),
            o16(), o16(), o16(),
        ],
        out_shape=[
            jax.ShapeDtypeStruct((B, P, OUT_CH), jnp.float32),
            s16(), s16(), s16(),
        ],
    )(y, stats, gamma, beta, wb_t, bb, wq_t, wk_t, wv_t)


PB_T = 128
RB_T = PB_T * K


def _attn_body(rows_ref, q_ref, cx_ref, np_ref,
               wp1_ref, bp1_ref, wp2_ref, bp2_ref,
               wa1_ref, ba1_ref, wa2_ref, ba2_ref, wa_ref, ba_ref, o_ref):
    rows = rows_ref[...]
    k_n = rows[:, :MID]
    v_n = rows[:, MID:2 * MID]
    x_n = rows[:, 2 * MID:2 * MID + 4]
    rel = (cx_ref[0].reshape(PB_T, 1, 4) - x_n.reshape(PB_T, K, 4)).reshape(RB_T, 4)
    rp = jnp.maximum(
        jnp.dot(rel, wp1_ref[...], preferred_element_type=jnp.float32) + bp1_ref[...], 0.0)
    rp = jnp.dot(rp, wp2_ref[...], preferred_element_type=jnp.float32) + bp2_ref[...]
    qk = (q_ref[0].reshape(PB_T, 1, MID) - k_n.reshape(PB_T, K, MID)).reshape(RB_T, MID)
    u = qk + rp
    s = jnp.maximum(
        jnp.dot(u, wa1_ref[...], preferred_element_type=jnp.float32) + ba1_ref[...], 0.0)
    s = jnp.dot(s, wa2_ref[...], preferred_element_type=jnp.float32) + ba2_ref[...]
    s3 = s.reshape(PB_T, K, MID)
    m = jnp.max(s3, axis=1, keepdims=True)
    e = jnp.exp(s3 - m)
    attn = e / jnp.sum(e, axis=1, keepdims=True)
    v2 = (v_n + rp).reshape(PB_T, K, MID)
    agg = jnp.sum(attn * v2, axis=1)
    o_ref[0] = np_ref[0] + jnp.dot(agg, wa_ref[...],
                                   preferred_element_type=jnp.float32) + ba_ref[...]


def _attn(g2rows, nxyz_rows, q, np_rows,
          wp1_t, bp1, wp2_t, bp2, wa1_t, ba1, wa2_t, ba2, wa_t, ba):
    grid = (B, P // PB_T)
    nb = P // PB_T
    w = lambda r, c: pl.BlockSpec((r, c), lambda b, p: (0, 0))
    return pl.pallas_call(
        _attn_body,
        grid=grid,
        in_specs=[
            pl.BlockSpec((RB_T, 48), lambda b, p: (b * nb + p, 0)),
            pl.BlockSpec((1, PB_T, MID), lambda b, p: (b, p, 0)),
            pl.BlockSpec((1, PB_T, 4), lambda b, p: (b, p, 0)),
            pl.BlockSpec((1, PB_T, OUT_CH), lambda b, p: (b, p, 0)),
            w(4, POS_H), w(1, POS_H), w(POS_H, MID), w(1, MID),
            w(MID, HID), w(1, HID), w(HID, MID), w(1, MID),
            w(MID, OUT_CH), w(1, OUT_CH),
        ],
        out_specs=pl.BlockSpec((1, PB_T, OUT_CH), lambda b, p: (b, p, 0)),
        out_shape=jax.ShapeDtypeStruct((B, P, OUT_CH), jnp.float32),
    )(g2rows, q, nxyz_rows, np_rows,
      wp1_t, bp1, wp2_t, bp2, wa1_t, ba1, wa2_t, ba2, wa_t, ba)


def kernel(xyz, points, W_down, b_down, bn_gamma, bn_beta, Wb, bb, Wqkv,
           Wp1, bp1, Wp2, bp2, Wa1, ba1, Wa2, ba2, Wa, ba):
    wf_t = jnp.transpose(W_down[:, :IN_CH])
    wx_t = jnp.concatenate(
        [jnp.transpose(W_down[:, IN_CH:]), jnp.zeros((1, OUT_CH))], axis=0)
    bd = b_down.reshape(1, OUT_CH)
    wb_t = jnp.transpose(Wb)
    wq_t = jnp.transpose(Wqkv[:MID])
    wk_t = jnp.transpose(Wqkv[MID:2 * MID])
    wv_t = jnp.transpose(Wqkv[2 * MID:])
    wp1_t = jnp.concatenate([jnp.transpose(Wp1), jnp.zeros((1, POS_H))], axis=0)
    wp2_t = jnp.transpose(Wp2)
    wa1_t = jnp.transpose(Wa1)
    wa2_t = jnp.transpose(Wa2)
    wa_t = jnp.transpose(Wa)

    new_xyz = _fps(xyz)
    nxyz_rows = jnp.concatenate(
        [jnp.transpose(new_xyz, (0, 2, 1)), jnp.zeros((B, P, 1))], axis=2)
    gidx1 = _knn(nxyz_rows, xyz, R=N)
    t1 = jnp.concatenate(
        [jnp.transpose(points, (0, 2, 1)),
         jnp.transpose(xyz, (0, 2, 1)),
         jnp.zeros((B, N, 13))], axis=2).reshape(B * N, 48)
    g1rows = _sc_gather48(t1, gidx1.reshape(-1))
    y, stats = _passa(g1rows, nxyz_rows, wf_t, wx_t, bd)
    np_rows, q, t2k, t2v = _passb(
        y, stats, bn_gamma.reshape(1, -1), bn_beta.reshape(1, -1),
        wb_t, bb.reshape(1, -1), wq_t, wk_t, wv_t)
    gidx2 = _knn(nxyz_rows, new_xyz, R=P)
    t2 = jnp.concatenate(
        [t2k, t2v, nxyz_rows, jnp.zeros((B, P, 12))], axis=2).reshape(B * P, 48)
    g2rows = _sc_gather48(t2, gidx2.reshape(-1))
    out_rows = _attn(g2rows, nxyz_rows, q, np_rows,
                     wp1_t, bp1.reshape(1, -1), wp2_t, bp2.reshape(1, -1),
                     wa1_t, ba1.reshape(1, -1), wa2_t, ba2.reshape(1, -1),
                     wa_t, ba.reshape(1, -1))
    return (new_xyz, jnp.transpose(out_rows, (0, 2, 1)))
